# baseline (speedup 1.0000x reference)
import functools

import jax
import jax.numpy as jnp
from jax import lax
from jax.experimental import pallas as pl
from jax.experimental.pallas import tpu as pltpu
from jax.experimental.pallas import tpu_sc as plsc

N = 10000
E = 320000
D_IN = 128
D_HID = 128
D_OUT = 40

NC = 2
NS = 16
NW = NC * NS
NPAD = 10240
ROWS_PER_TILE = NPAD // NS
RPT = 80
EPAD = NW * RPT * 128

_MESH = plsc.VectorSubcoreMesh(
    core_axis_name="c", subcore_axis_name="s", num_cores=NC, num_subcores=NS
)


def _deg_kernel_body(dst_hbm, deg_hbm, dst_v, deg_v):
  c = lax.axis_index("c")
  s = lax.axis_index("s")
  wid = s * NC + c
  pltpu.sync_copy(dst_hbm.at[wid], dst_v)

  def zero_body(i, _):
    deg_v[pl.ds(i * 16, 16)] = jnp.zeros((16,), jnp.float32)
    return ()
  lax.fori_loop(0, NPAD // 16, zero_body, ())

  ones = jnp.ones((16,), jnp.float32)

  def acc_body(i, _):
    r = i // 8
    k = i % 8
    idx = dst_v[r, pl.ds(k * 16, 16)]
    plsc.addupdate_scatter(deg_v, [idx], ones)
    return ()
  lax.fori_loop(0, RPT * 8, acc_body, ())

  pltpu.sync_copy(deg_v, deg_hbm.at[wid])


_deg_kernel = functools.partial(
    pl.kernel,
    out_type=jax.ShapeDtypeStruct((NW, NPAD), jnp.float32),
    mesh=_MESH,
    scratch_types=[
        pltpu.VMEM((RPT, 128), jnp.int32),
        pltpu.VMEM((NPAD,), jnp.float32),
    ],
    compiler_params=pltpu.CompilerParams(needs_layout_passes=False),
)(_deg_kernel_body)


NBUF = 2
RPH = RPT // 2


def _make_agg_kernel(D):
  nseg = D // 16

  def body(src_hbm, dst_hbm, xs_hbm, out_hbm, src_v, dst_v, gbuf, agg_sh,
           gsem00, gsem01, gsem10, gsem11):
    c = lax.axis_index("c")
    s = lax.axis_index("s")
    wid = s * NC + c

    def zero_body(i, _):
      r = i // nseg
      k = i % nseg
      gbuf[0, r, pl.ds(k * 16, 16)] = jnp.zeros((16,), jnp.float32)
      return ()
    lax.fori_loop(0, 128 * nseg, zero_body, ())
    for r in range(ROWS_PER_TILE // 128):
      pltpu.sync_copy(
          gbuf.at[0], agg_sh.at[pl.ds(s * ROWS_PER_TILE + r * 128, 128)]
      )
    plsc.subcore_barrier()

    sems = ((gsem00, gsem01), (gsem10, gsem11))

    def gath(i, b):
      for q in range(2):
        pltpu.async_copy(
            xs_hbm.at[src_v.at[i, pl.ds(q * 64, 64)]],
            gbuf.at[b, pl.ds(q * 64, 64)],
            sems[b][q],
        )

    def gwait(i, b):
      for q in range(2):
        pltpu.make_async_copy(
            xs_hbm.at[src_v.at[i, pl.ds(q * 64, 64)]],
            gbuf.at[b, pl.ds(q * 64, 64)],
            sems[b][q],
        ).wait()

    for h in range(2):
      pltpu.sync_copy(src_hbm.at[wid, pl.ds(h * RPH, RPH)], src_v)
      pltpu.sync_copy(dst_hbm.at[wid, pl.ds(h * RPH, RPH)], dst_v)

      for b in range(NBUF):
        gath(b, b)

      def jbody(j, _):
        i0 = j * NBUF
        for b in range(NBUF):
          gwait(i0 + b, b)
          pltpu.sync_copy(gbuf.at[b], agg_sh.at[dst_v.at[i0 + b]], add=True)
          gath(i0 + NBUF + b, b)
        return ()
      lax.fori_loop(0, RPH // NBUF - 1, jbody, ())

      tail = RPH - NBUF
      for b in range(NBUF):
        gwait(tail + b, b)
        pltpu.sync_copy(gbuf.at[b], agg_sh.at[dst_v.at[tail + b]], add=True)

    plsc.subcore_barrier()
    pltpu.sync_copy(
        agg_sh.at[pl.ds(s * ROWS_PER_TILE, ROWS_PER_TILE)],
        out_hbm.at[c, pl.ds(s * ROWS_PER_TILE, ROWS_PER_TILE)],
    )

  return functools.partial(
      pl.kernel,
      out_type=jax.ShapeDtypeStruct((NC, NPAD, D), jnp.float32),
      mesh=_MESH,
      scratch_types=[
          pltpu.VMEM((RPH, 128), jnp.int32),
          pltpu.VMEM((RPH, 128), jnp.int32),
          pltpu.VMEM((NBUF, 128, D), jnp.float32),
          pltpu.VMEM_SHARED((NPAD, D), jnp.float32),
          pltpu.SemaphoreType.DMA,
          pltpu.SemaphoreType.DMA,
          pltpu.SemaphoreType.DMA,
          pltpu.SemaphoreType.DMA,
      ],
      compiler_params=pltpu.CompilerParams(needs_layout_passes=False),
  )(body)


_agg128 = _make_agg_kernel(D_HID)


_R = 1024


def _tc1(degp_t, x, w1):
  def body(deg_ref, x_ref, w_ref, dinv_ref, xs_ref):
    dsum = jnp.sum(deg_ref[...], axis=1, keepdims=True) + 1.0
    dinv = lax.rsqrt(dsum)
    dinv_ref[...] = dinv
    xs_ref[...] = dinv * jnp.dot(
        x_ref[...], w_ref[...], preferred_element_type=jnp.float32
    )

  return pl.pallas_call(
      body,
      grid=(NPAD // _R,),
      in_specs=[
          pl.BlockSpec((_R, NW), lambda i: (i, 0)),
          pl.BlockSpec((_R, D_IN), lambda i: (i, 0)),
          pl.BlockSpec((D_IN, D_HID), lambda i: (0, 0)),
      ],
      out_specs=[
          pl.BlockSpec((_R, 1), lambda i: (i, 0)),
          pl.BlockSpec((_R, D_HID), lambda i: (i, 0)),
      ],
      out_shape=[
          jax.ShapeDtypeStruct((NPAD, 1), jnp.float32),
          jax.ShapeDtypeStruct((NPAD, D_HID), jnp.float32),
      ],
  )(degp_t, x, w1)


def _tc2(agg1p, xs1, dinv, b1):
  def body(aggp_ref, xs1_ref, dinv_ref, b1_ref, hs_ref):
    a = aggp_ref[0] + aggp_ref[1] + xs1_ref[...]
    dinv = dinv_ref[...]
    h = jnp.maximum(dinv * a + b1_ref[...], 0.0)
    hs_ref[...] = dinv * h

  return pl.pallas_call(
      body,
      grid=(NPAD // _R,),
      in_specs=[
          pl.BlockSpec((NC, _R, D_HID), lambda i: (0, i, 0)),
          pl.BlockSpec((_R, D_HID), lambda i: (i, 0)),
          pl.BlockSpec((_R, 1), lambda i: (i, 0)),
          pl.BlockSpec((1, D_HID), lambda i: (0, 0)),
      ],
      out_specs=pl.BlockSpec((_R, D_HID), lambda i: (i, 0)),
      out_shape=jax.ShapeDtypeStruct((NPAD, D_HID), jnp.float32),
  )(agg1p, xs1, dinv, b1)


def _tc3(agg2p, hs, dinv, b2, w2):
  def body(aggp_ref, hs_ref, dinv_ref, b2_ref, w2_ref, y_ref):
    a = dinv_ref[...] * (aggp_ref[0] + aggp_ref[1] + hs_ref[...])
    y_ref[...] = (
        jnp.dot(a, w2_ref[...], preferred_element_type=jnp.float32)
        + b2_ref[...]
    )

  return pl.pallas_call(
      body,
      grid=(NPAD // _R,),
      in_specs=[
          pl.BlockSpec((NC, _R, D_HID), lambda i: (0, i, 0)),
          pl.BlockSpec((_R, D_HID), lambda i: (i, 0)),
          pl.BlockSpec((_R, 1), lambda i: (i, 0)),
          pl.BlockSpec((1, D_OUT), lambda i: (0, 0)),
          pl.BlockSpec((D_HID, D_OUT), lambda i: (0, 0)),
      ],
      out_specs=pl.BlockSpec((_R, D_OUT), lambda i: (i, 0)),
      out_shape=jax.ShapeDtypeStruct((NPAD, D_OUT), jnp.float32),
  )(agg2p, hs, dinv, b2, w2)


def kernel(x, edge_index, W1, b1, W2, b2):
  src = edge_index[0].astype(jnp.int32)
  dst = edge_index[1].astype(jnp.int32)
  npad_e = EPAD - E
  src_p = jnp.concatenate([src, jnp.zeros((npad_e,), jnp.int32)])
  dst_p = jnp.concatenate([dst, jnp.full((npad_e,), N, jnp.int32)])
  src_p = src_p.reshape(NW, RPT, 128)
  dst_p = dst_p.reshape(NW, RPT, 128)
  x_p = jnp.pad(x, ((0, NPAD - N), (0, 0)))

  degp = _deg_kernel(dst_p)
  dinv, xs1 = _tc1(degp.T, x_p, W1)
  agg1p = _agg128(src_p, dst_p, xs1)
  hs = _tc2(agg1p, xs1, dinv, b1.reshape(1, D_HID))
  agg2p = _agg128(src_p, dst_p, hs)
  y = _tc3(agg2p, hs, dinv, b2.reshape(1, D_OUT), W2)
  return y[:N]


# speedup vs baseline: 1.0024x; 1.0024x over previous
import functools

import jax
import jax.numpy as jnp
from jax import lax
from jax.experimental import pallas as pl
from jax.experimental.pallas import tpu as pltpu
from jax.experimental.pallas import tpu_sc as plsc

N = 10000
E = 320000
D_IN = 128
D_HID = 128
D_OUT = 40

NC = 2
NS = 16
NW = NC * NS
NPAD = 10240
ROWS_PER_TILE = NPAD // NS
RPT = 80
EPAD = NW * RPT * 128

_MESH = plsc.VectorSubcoreMesh(
    core_axis_name="c", subcore_axis_name="s", num_cores=NC, num_subcores=NS
)


def _deg_kernel_body(dst_hbm, deg_hbm, dst_v, deg_v):
  c = lax.axis_index("c")
  s = lax.axis_index("s")
  wid = s * NC + c
  pltpu.sync_copy(dst_hbm.at[wid], dst_v)

  def zero_body(i, _):
    deg_v[pl.ds(i * 16, 16)] = jnp.zeros((16,), jnp.float32)
    return ()
  lax.fori_loop(0, NPAD // 16, zero_body, ())

  ones = jnp.ones((16,), jnp.float32)

  def acc_body(i, _):
    r = i // 8
    k = i % 8
    idx = dst_v[r, pl.ds(k * 16, 16)]
    plsc.addupdate_scatter(deg_v, [idx], ones)
    return ()
  lax.fori_loop(0, RPT * 8, acc_body, ())

  pltpu.sync_copy(deg_v, deg_hbm.at[wid])


_deg_kernel = functools.partial(
    pl.kernel,
    out_type=jax.ShapeDtypeStruct((NW, NPAD), jnp.float32),
    mesh=_MESH,
    scratch_types=[
        pltpu.VMEM((RPT, 128), jnp.int32),
        pltpu.VMEM((NPAD,), jnp.float32),
    ],
    compiler_params=pltpu.CompilerParams(needs_layout_passes=False),
)(_deg_kernel_body)


NBUF = 2
RPH = RPT // 2


def _make_agg_kernel(D):
  nseg = D // 16

  def body(src_hbm, dst_hbm, xs_hbm, out_hbm, src_v, dst_v, gbuf, agg_sh,
           gsem00, gsem10):
    c = lax.axis_index("c")
    s = lax.axis_index("s")
    wid = s * NC + c

    def zero_body(i, _):
      r = i // nseg
      k = i % nseg
      gbuf[0, r, pl.ds(k * 16, 16)] = jnp.zeros((16,), jnp.float32)
      return ()
    lax.fori_loop(0, 128 * nseg, zero_body, ())
    for r in range(ROWS_PER_TILE // 128):
      pltpu.sync_copy(
          gbuf.at[0], agg_sh.at[pl.ds(s * ROWS_PER_TILE + r * 128, 128)]
      )
    plsc.subcore_barrier()

    sems = (gsem00, gsem10)
    for h in range(2):
      pltpu.sync_copy(src_hbm.at[wid, pl.ds(h * RPH, RPH)], src_v)
      pltpu.sync_copy(dst_hbm.at[wid, pl.ds(h * RPH, RPH)], dst_v)

      for b in range(NBUF):
        pltpu.async_copy(xs_hbm.at[src_v.at[b]], gbuf.at[b], sems[b])

      def jbody(j, _):
        i0 = j * NBUF
        for b in range(NBUF):
          pltpu.make_async_copy(
              xs_hbm.at[src_v.at[i0 + b]], gbuf.at[b], sems[b]
          ).wait()
          pltpu.sync_copy(gbuf.at[b], agg_sh.at[dst_v.at[i0 + b]], add=True)
          pltpu.async_copy(
              xs_hbm.at[src_v.at[i0 + NBUF + b]], gbuf.at[b], sems[b]
          )
        return ()
      lax.fori_loop(0, RPH // NBUF - 1, jbody, ())

      tail = RPH - NBUF
      for b in range(NBUF):
        pltpu.make_async_copy(
            xs_hbm.at[src_v.at[tail + b]], gbuf.at[b], sems[b]
        ).wait()
        pltpu.sync_copy(gbuf.at[b], agg_sh.at[dst_v.at[tail + b]], add=True)

    plsc.subcore_barrier()
    pltpu.sync_copy(
        agg_sh.at[pl.ds(s * ROWS_PER_TILE, ROWS_PER_TILE)],
        out_hbm.at[c, pl.ds(s * ROWS_PER_TILE, ROWS_PER_TILE)],
    )

  return functools.partial(
      pl.kernel,
      out_type=jax.ShapeDtypeStruct((NC, NPAD, D), jnp.float32),
      mesh=_MESH,
      scratch_types=[
          pltpu.VMEM((RPH, 128), jnp.int32),
          pltpu.VMEM((RPH, 128), jnp.int32),
          pltpu.VMEM((NBUF, 128, D), jnp.float32),
          pltpu.VMEM_SHARED((NPAD, D), jnp.float32),
          pltpu.SemaphoreType.DMA,
          pltpu.SemaphoreType.DMA,
      ],
      compiler_params=pltpu.CompilerParams(needs_layout_passes=False),
  )(body)


_agg128 = _make_agg_kernel(D_HID)


_R = 1024


def _tc_mm(x, w1):
  def body(x_ref, w_ref, xw_ref):
    xw_ref[...] = jnp.dot(
        x_ref[...], w_ref[...], preferred_element_type=jnp.float32
    )

  return pl.pallas_call(
      body,
      grid=(NPAD // _R,),
      in_specs=[
          pl.BlockSpec((_R, D_IN), lambda i: (i, 0)),
          pl.BlockSpec((D_IN, D_HID), lambda i: (0, 0)),
      ],
      out_specs=pl.BlockSpec((_R, D_HID), lambda i: (i, 0)),
      out_shape=jax.ShapeDtypeStruct((NPAD, D_HID), jnp.float32),
  )(x, w1)


def _tc1(degp_t, xw):
  def body(deg_ref, xw_ref, dinv_ref, xs_ref):
    dsum = jnp.sum(deg_ref[...], axis=1, keepdims=True) + 1.0
    dinv = lax.rsqrt(dsum)
    dinv_ref[...] = dinv
    xs_ref[...] = dinv * xw_ref[...]

  return pl.pallas_call(
      body,
      grid=(NPAD // _R,),
      in_specs=[
          pl.BlockSpec((_R, NW), lambda i: (i, 0)),
          pl.BlockSpec((_R, D_HID), lambda i: (i, 0)),
      ],
      out_specs=[
          pl.BlockSpec((_R, 1), lambda i: (i, 0)),
          pl.BlockSpec((_R, D_HID), lambda i: (i, 0)),
      ],
      out_shape=[
          jax.ShapeDtypeStruct((NPAD, 1), jnp.float32),
          jax.ShapeDtypeStruct((NPAD, D_HID), jnp.float32),
      ],
  )(degp_t, xw)


def _tc2(agg1p, xs1, dinv, b1):
  def body(aggp_ref, xs1_ref, dinv_ref, b1_ref, hs_ref):
    a = aggp_ref[0] + aggp_ref[1] + xs1_ref[...]
    dinv = dinv_ref[...]
    h = jnp.maximum(dinv * a + b1_ref[...], 0.0)
    hs_ref[...] = dinv * h

  return pl.pallas_call(
      body,
      grid=(NPAD // _R,),
      in_specs=[
          pl.BlockSpec((NC, _R, D_HID), lambda i: (0, i, 0)),
          pl.BlockSpec((_R, D_HID), lambda i: (i, 0)),
          pl.BlockSpec((_R, 1), lambda i: (i, 0)),
          pl.BlockSpec((1, D_HID), lambda i: (0, 0)),
      ],
      out_specs=pl.BlockSpec((_R, D_HID), lambda i: (i, 0)),
      out_shape=jax.ShapeDtypeStruct((NPAD, D_HID), jnp.float32),
  )(agg1p, xs1, dinv, b1)


def _tc3(agg2p, hs, dinv, b2, w2):
  def body(aggp_ref, hs_ref, dinv_ref, b2_ref, w2_ref, y_ref):
    a = dinv_ref[...] * (aggp_ref[0] + aggp_ref[1] + hs_ref[...])
    y_ref[...] = (
        jnp.dot(a, w2_ref[...], preferred_element_type=jnp.float32)
        + b2_ref[...]
    )

  return pl.pallas_call(
      body,
      grid=(NPAD // _R,),
      in_specs=[
          pl.BlockSpec((NC, _R, D_HID), lambda i: (0, i, 0)),
          pl.BlockSpec((_R, D_HID), lambda i: (i, 0)),
          pl.BlockSpec((_R, 1), lambda i: (i, 0)),
          pl.BlockSpec((1, D_OUT), lambda i: (0, 0)),
          pl.BlockSpec((D_HID, D_OUT), lambda i: (0, 0)),
      ],
      out_specs=pl.BlockSpec((_R, D_OUT), lambda i: (i, 0)),
      out_shape=jax.ShapeDtypeStruct((NPAD, D_OUT), jnp.float32),
  )(agg2p, hs, dinv, b2, w2)


def kernel(x, edge_index, W1, b1, W2, b2):
  src = edge_index[0].astype(jnp.int32)
  dst = edge_index[1].astype(jnp.int32)
  npad_e = EPAD - E
  src_p = jnp.concatenate([src, jnp.zeros((npad_e,), jnp.int32)])
  dst_p = jnp.concatenate([dst, jnp.full((npad_e,), N, jnp.int32)])
  src_p = src_p.reshape(NW, RPT, 128)
  dst_p = dst_p.reshape(NW, RPT, 128)
  x_p = jnp.pad(x, ((0, NPAD - N), (0, 0)))

  degp = _deg_kernel(dst_p)
  xw = _tc_mm(x_p, W1)
  dinv, xs1 = _tc1(degp.T, xw)
  agg1p = _agg128(src_p, dst_p, xs1)
  hs = _tc2(agg1p, xs1, dinv, b1.reshape(1, D_HID))
  agg2p = _agg128(src_p, dst_p, hs)
  y = _tc3(agg2p, hs, dinv, b2.reshape(1, D_OUT), W2)
  return y[:N]
